# baseline (speedup 1.0000x reference)
import functools

import jax
import jax.numpy as jnp
from jax import lax
from jax.experimental import pallas as pl
from jax.experimental.pallas import tpu as pltpu
from jax.experimental.pallas import tpu_sc as plsc

N = 10000
E = 320000
D = 128
NC, NS = 2, 16
NW = NC * NS
EPW = E // NW
K = 100
NCHUNK = EPW // K
SCH = 25
NSEC = NCHUNK // SCH
NPAD = 10240
RPT = NPAD // NS
ZB = 80


def _sc_aggregate(x, row4, col4):
    mesh = plsc.VectorSubcoreMesh(core_axis_name="c", subcore_axis_name="s")

    @functools.partial(
        pl.kernel,
        out_type=(
            jax.ShapeDtypeStruct((NC, NPAD, D), jnp.float32),
            jax.ShapeDtypeStruct((NC, NPAD), jnp.float32),
        ),
        mesh=mesh,
        scratch_types=[
            pltpu.VMEM_SHARED((NPAD, D), jnp.float32),
            pltpu.VMEM_SHARED((NPAD,), jnp.float32),
            pltpu.VMEM((SCH, K), jnp.int32),
            pltpu.VMEM((SCH, K), jnp.int32),
            pltpu.VMEM((K, D), jnp.float32),
            pltpu.VMEM((K, D), jnp.float32),
            pltpu.VMEM((K,), jnp.float32),
            pltpu.VMEM((RPT,), jnp.float32),
            pltpu.SemaphoreType.DMA,
            pltpu.SemaphoreType.DMA,
            pltpu.SemaphoreType.DMA,
            pltpu.SemaphoreType.DMA,
        ],
    )
    def agg_kernel(x_hbm, row_hbm, col_hbm, ones_hbm, agg_hbm, cnt_hbm,
                   agg_s, cnt_s, ridx, cidx, rows0, rows1, ones, zcnt,
                   sem0, sem1, sem2, sem3):
        c = lax.axis_index("c")
        s = lax.axis_index("s")
        wid = s * NC + c

        zv = jnp.zeros((16,), jnp.float32)

        pltpu.sync_copy(ones_hbm, ones)

        def _zc(i, carry):
            zcnt[pl.ds(i * 16, 16)] = zv
            return carry
        lax.fori_loop(0, RPT // 16, _zc, 0)

        def _zb(i, carry):
            for j in range(D // 16):
                rows0[i, pl.ds(j * 16, 16)] = zv
            return carry
        lax.fori_loop(0, K, _zb, 0)

        base_r = s * RPT
        for jb in range(RPT // ZB):
            pltpu.sync_copy(rows0.at[pl.ds(0, ZB), :],
                            agg_s.at[pl.ds(base_r + jb * ZB, ZB), :])
        pltpu.sync_copy(zcnt, cnt_s.at[pl.ds(base_r, RPT)])

        plsc.subcore_barrier()

        def _g_start(j, buf, sem):
            pltpu.async_copy(x_hbm.at[cidx.at[j]], buf, sem)

        def _g_wait(j, buf, sem):
            pltpu.make_async_copy(x_hbm.at[cidx.at[j]], buf, sem).wait()

        def _s_start(j, buf, sem):
            pltpu.async_copy(buf, agg_s.at[ridx.at[j]], sem, add=True)
            pltpu.sync_copy(ones, cnt_s.at[ridx.at[j]], add=True)

        def _s_wait(j, buf, sem):
            pltpu.make_async_copy(buf, agg_s.at[ridx.at[j]], sem).wait()

        def _sec(sec, carry):
            pltpu.sync_copy(row_hbm.at[wid, sec], ridx)
            pltpu.sync_copy(col_hbm.at[wid, sec], cidx)

            _g_start(0, rows0, sem0)

            def _pair(t, carry2):
                j0 = t * 2
                _g_start(j0 + 1, rows1, sem1)
                _g_wait(j0, rows0, sem0)
                _s_start(j0, rows0, sem2)
                _g_wait(j0 + 1, rows1, sem1)
                _s_start(j0 + 1, rows1, sem3)
                _s_wait(j0, rows0, sem2)
                _g_start(j0 + 2, rows0, sem0)
                _s_wait(j0 + 1, rows1, sem3)
                return carry2
            lax.fori_loop(0, (SCH - 1) // 2, _pair, 0)

            _g_wait(SCH - 1, rows0, sem0)
            pltpu.sync_copy(rows0, agg_s.at[ridx.at[SCH - 1]], add=True)
            pltpu.sync_copy(ones, cnt_s.at[ridx.at[SCH - 1]], add=True)
            return carry
        lax.fori_loop(0, NSEC, _sec, 0)

        plsc.subcore_barrier()

        pltpu.sync_copy(agg_s.at[pl.ds(base_r, RPT), :],
                        agg_hbm.at[c, pl.ds(base_r, RPT), :])
        pltpu.sync_copy(cnt_s.at[pl.ds(base_r, RPT)],
                        cnt_hbm.at[c, pl.ds(base_r, RPT)])

    return agg_kernel(x, row4, col4, jnp.ones((K,), jnp.float32))


def _tc_body(x_ref, agg_ref, cnt_ref, w_ref, b_ref, g_ref, be_ref, out_ref):
    x = x_ref[...]
    agg = agg_ref[0, :N, :] + agg_ref[1, :N, :]
    cnt = cnt_ref[0, :N, :] + cnt_ref[1, :N, :]
    inv = 1.0 / (cnt + 1e-8)
    u = x + agg * inv
    y = lax.dot_general(u, w_ref[...], (((1,), (1,)), ((), ())),
                        preferred_element_type=jnp.float32)
    y = y + (1.0 + cnt * inv) * b_ref[...]
    mean = jnp.mean(y, axis=0, keepdims=True)
    yc = y - mean
    var = jnp.mean(yc * yc, axis=0, keepdims=True)
    out_ref[...] = yc * lax.rsqrt(var + 1e-5) * g_ref[...] + be_ref[...]


def kernel(x, edge_index, batch_size, W, b, gamma, beta):
    del batch_size
    ei = edge_index.astype(jnp.int32)
    row4 = ei[0].reshape(NW, NSEC, SCH, K)
    col4 = ei[1].reshape(NW, NSEC, SCH, K)
    agg, cnt = _sc_aggregate(x, row4, col4)
    cnt3 = cnt.reshape(NC, NPAD, 1)
    out = pl.pallas_call(
        _tc_body,
        out_shape=jax.ShapeDtypeStruct((N, D), jnp.float32),
    )(x, agg, cnt3, W, b.reshape(1, D), gamma.reshape(1, D),
      beta.reshape(1, D))
    return out


# speedup vs baseline: 1.0244x; 1.0244x over previous
import functools

import jax
import jax.numpy as jnp
from jax import lax
from jax.experimental import pallas as pl
from jax.experimental.pallas import tpu as pltpu
from jax.experimental.pallas import tpu_sc as plsc

N = 10000
E = 320000
D = 128
NC, NS = 2, 16
NW = NC * NS
EPW = E // NW
K = 125
NCHUNK = EPW // K
SCH = 5
NSEC = NCHUNK // SCH
NPAD = 10240
RPT = NPAD // NS
ZB = 80


def _sc_aggregate(x, row4, col4):
    mesh = plsc.VectorSubcoreMesh(core_axis_name="c", subcore_axis_name="s")

    @functools.partial(
        pl.kernel,
        out_type=(
            jax.ShapeDtypeStruct((NC, NPAD, D), jnp.float32),
            jax.ShapeDtypeStruct((NC, NPAD), jnp.float32),
        ),
        mesh=mesh,
        scratch_types=[
            pltpu.VMEM_SHARED((NPAD, D), jnp.float32),
            pltpu.VMEM_SHARED((NPAD,), jnp.float32),
            pltpu.VMEM((SCH, K), jnp.int32),
            pltpu.VMEM((SCH, K), jnp.int32),
            pltpu.VMEM((K, D), jnp.float32),
            pltpu.VMEM((K, D), jnp.float32),
            pltpu.VMEM((K,), jnp.float32),
            pltpu.VMEM((RPT,), jnp.float32),
            pltpu.SemaphoreType.DMA,
            pltpu.SemaphoreType.DMA,
        ],
    )
    def agg_kernel(x_hbm, row_hbm, col_hbm, ones_hbm, agg_hbm, cnt_hbm,
                   agg_s, cnt_s, ridx, cidx, rows0, rows1, ones, zcnt,
                   sem0, sem1):
        c = lax.axis_index("c")
        s = lax.axis_index("s")
        wid = s * NC + c

        zv = jnp.zeros((16,), jnp.float32)

        pltpu.sync_copy(ones_hbm, ones)

        def _zc(i, carry):
            zcnt[pl.ds(i * 16, 16)] = zv
            return carry
        lax.fori_loop(0, RPT // 16, _zc, 0)

        def _zb(i, carry):
            for j in range(D // 16):
                rows0[i, pl.ds(j * 16, 16)] = zv
            return carry
        lax.fori_loop(0, K, _zb, 0)

        base_r = s * RPT
        for jb in range(RPT // ZB):
            pltpu.sync_copy(rows0.at[pl.ds(0, ZB), :],
                            agg_s.at[pl.ds(base_r + jb * ZB, ZB), :])
        pltpu.sync_copy(zcnt, cnt_s.at[pl.ds(base_r, RPT)])

        plsc.subcore_barrier()

        def _g_start(j, buf, sem):
            pltpu.async_copy(x_hbm.at[cidx.at[j]], buf, sem)

        def _g_wait(j, buf, sem):
            pltpu.make_async_copy(x_hbm.at[cidx.at[j]], buf, sem).wait()

        def _scat(j, buf):
            pltpu.sync_copy(buf, agg_s.at[ridx.at[j]], add=True)
            pltpu.sync_copy(ones, cnt_s.at[ridx.at[j]], add=True)

        def _sec(sec, carry):
            pltpu.sync_copy(row_hbm.at[wid, sec], ridx)
            pltpu.sync_copy(col_hbm.at[wid, sec], cidx)

            _g_start(0, rows0, sem0)

            def _pair(t, carry2):
                j0 = t * 2
                _g_start(j0 + 1, rows1, sem1)
                _g_wait(j0, rows0, sem0)
                _scat(j0, rows0)
                _g_start(j0 + 2, rows0, sem0)
                _g_wait(j0 + 1, rows1, sem1)
                _scat(j0 + 1, rows1)
                return carry2
            lax.fori_loop(0, (SCH - 1) // 2, _pair, 0)

            _g_wait(SCH - 1, rows0, sem0)
            _scat(SCH - 1, rows0)
            return carry
        lax.fori_loop(0, NSEC, _sec, 0)

        plsc.subcore_barrier()

        pltpu.sync_copy(agg_s.at[pl.ds(base_r, RPT), :],
                        agg_hbm.at[c, pl.ds(base_r, RPT), :])
        pltpu.sync_copy(cnt_s.at[pl.ds(base_r, RPT)],
                        cnt_hbm.at[c, pl.ds(base_r, RPT)])

    return agg_kernel(x, row4, col4, jnp.ones((K,), jnp.float32))


def _tc_body(x_ref, agg_ref, cnt_ref, w_ref, b_ref, g_ref, be_ref, out_ref):
    x = x_ref[...]
    agg = agg_ref[0, :N, :] + agg_ref[1, :N, :]
    cnt = cnt_ref[0, :N, :] + cnt_ref[1, :N, :]
    inv = 1.0 / (cnt + 1e-8)
    u = x + agg * inv
    y = lax.dot_general(u, w_ref[...], (((1,), (1,)), ((), ())),
                        preferred_element_type=jnp.float32)
    y = y + (1.0 + cnt * inv) * b_ref[...]
    mean = jnp.mean(y, axis=0, keepdims=True)
    yc = y - mean
    var = jnp.mean(yc * yc, axis=0, keepdims=True)
    out_ref[...] = yc * lax.rsqrt(var + 1e-5) * g_ref[...] + be_ref[...]


def kernel(x, edge_index, batch_size, W, b, gamma, beta):
    del batch_size
    ei = edge_index.astype(jnp.int32)
    row4 = ei[0].reshape(NW, NSEC, SCH, K)
    col4 = ei[1].reshape(NW, NSEC, SCH, K)
    agg, cnt = _sc_aggregate(x, row4, col4)
    cnt3 = cnt.reshape(NC, NPAD, 1)
    out = pl.pallas_call(
        _tc_body,
        out_shape=jax.ShapeDtypeStruct((N, D), jnp.float32),
    )(x, agg, cnt3, W, b.reshape(1, D), gamma.reshape(1, D),
      beta.reshape(1, D))
    return out


# speedup vs baseline: 1.1345x; 1.1075x over previous
import functools

import jax
import jax.numpy as jnp
from jax import lax
from jax.experimental import pallas as pl
from jax.experimental.pallas import tpu as pltpu
from jax.experimental.pallas import tpu_sc as plsc

N = 10000
E = 320000
D = 128
NC, NS = 2, 16
NW = NC * NS
EPW = E // NW
K = 100
NCHUNK = EPW // K
SCH = 25
NSEC = NCHUNK // SCH
NPAD = 10240
RPT = NPAD // NS
ZB = 80


def _sc_aggregate(x, row4, col4):
    mesh = plsc.VectorSubcoreMesh(core_axis_name="c", subcore_axis_name="s")

    @functools.partial(
        pl.kernel,
        out_type=(
            jax.ShapeDtypeStruct((NC, NPAD, D), jnp.float32),
            jax.ShapeDtypeStruct((NC, NPAD), jnp.float32),
        ),
        mesh=mesh,
        scratch_types=[
            pltpu.VMEM_SHARED((NPAD, D), jnp.float32),
            pltpu.VMEM_SHARED((NPAD,), jnp.float32),
            pltpu.VMEM((SCH, K), jnp.int32),
            pltpu.VMEM((SCH, K), jnp.int32),
            pltpu.VMEM((K, D), jnp.float32),
            pltpu.VMEM((K, D), jnp.float32),
            pltpu.VMEM((K,), jnp.float32),
            pltpu.VMEM((RPT,), jnp.float32),
            pltpu.SemaphoreType.DMA,
            pltpu.SemaphoreType.DMA,
        ],
    )
    def agg_kernel(x_hbm, row_hbm, col_hbm, ones_hbm, agg_hbm, cnt_hbm,
                   agg_s, cnt_s, ridx, cidx, rows0, rows1, ones, zcnt,
                   sem0, sem1):
        c = lax.axis_index("c")
        s = lax.axis_index("s")
        wid = s * NC + c

        zv = jnp.zeros((16,), jnp.float32)

        pltpu.sync_copy(ones_hbm, ones)

        def _zc(i, carry):
            zcnt[pl.ds(i * 16, 16)] = zv
            return carry
        lax.fori_loop(0, RPT // 16, _zc, 0)

        def _zb(i, carry):
            for j in range(D // 16):
                rows0[i, pl.ds(j * 16, 16)] = zv
            return carry
        lax.fori_loop(0, K, _zb, 0)

        base_r = s * RPT
        for jb in range(RPT // ZB):
            pltpu.sync_copy(rows0.at[pl.ds(0, ZB), :],
                            agg_s.at[pl.ds(base_r + jb * ZB, ZB), :])
        pltpu.sync_copy(zcnt, cnt_s.at[pl.ds(base_r, RPT)])

        plsc.subcore_barrier()

        def _g_start(j, buf, sem):
            pltpu.async_copy(x_hbm.at[cidx.at[j]], buf, sem)

        def _g_wait(j, buf, sem):
            pltpu.make_async_copy(x_hbm.at[cidx.at[j]], buf, sem).wait()

        def _scat(j, buf):
            pltpu.sync_copy(buf, agg_s.at[ridx.at[j]], add=True)
            pltpu.sync_copy(ones, cnt_s.at[ridx.at[j]], add=True)

        def _sec(sec, carry):
            pltpu.sync_copy(row_hbm.at[wid, sec], ridx)
            pltpu.sync_copy(col_hbm.at[wid, sec], cidx)

            _g_start(0, rows0, sem0)

            def _pair(t, carry2):
                j0 = t * 2
                _g_start(j0 + 1, rows1, sem1)
                _g_wait(j0, rows0, sem0)
                _scat(j0, rows0)
                _g_start(j0 + 2, rows0, sem0)
                _g_wait(j0 + 1, rows1, sem1)
                _scat(j0 + 1, rows1)
                return carry2
            lax.fori_loop(0, (SCH - 1) // 2, _pair, 0)

            _g_wait(SCH - 1, rows0, sem0)
            _scat(SCH - 1, rows0)
            return carry
        lax.fori_loop(0, NSEC, _sec, 0)

        plsc.subcore_barrier()

        pltpu.sync_copy(agg_s.at[pl.ds(base_r, RPT), :],
                        agg_hbm.at[c, pl.ds(base_r, RPT), :])
        pltpu.sync_copy(cnt_s.at[pl.ds(base_r, RPT)],
                        cnt_hbm.at[c, pl.ds(base_r, RPT)])

    return agg_kernel(x, row4, col4, jnp.ones((K,), jnp.float32))


BR = 1000
NB = N // BR


def _tc_body(x_ref, agg_ref, cnt_ref, w_ref, b_ref, g_ref, be_ref, out_ref,
             y_s, ssum, ssq):
    p = pl.program_id(0)
    i = pl.program_id(1)

    @pl.when(p == 0)
    def _phase0():
        @pl.when(i == 0)
        def _init():
            ssum[...] = jnp.zeros_like(ssum)
            ssq[...] = jnp.zeros_like(ssq)
        x = x_ref[...]
        agg = agg_ref[0] + agg_ref[1]
        cnt = cnt_ref[0] + cnt_ref[1]
        inv = 1.0 / (cnt + 1e-8)
        u = x + agg * inv
        y = lax.dot_general(u, w_ref[...], (((1,), (1,)), ((), ())),
                            preferred_element_type=jnp.float32)
        y = y + (1.0 + cnt * inv) * b_ref[...]
        y_s[pl.ds(i * BR, BR), :] = y
        ssum[...] += jnp.sum(y, axis=0, keepdims=True)
        ssq[...] += jnp.sum(y * y, axis=0, keepdims=True)

    @pl.when(p == 1)
    def _phase1():
        mean = ssum[...] * (1.0 / N)
        var = ssq[...] * (1.0 / N) - mean * mean
        scale = lax.rsqrt(var + 1e-5) * g_ref[...]
        y = y_s[pl.ds(i * BR, BR), :]
        out_ref[...] = (y - mean) * scale + be_ref[...]


def kernel(x, edge_index, batch_size, W, b, gamma, beta):
    del batch_size
    ei = edge_index.astype(jnp.int32)
    row4 = ei[0].reshape(NW, NSEC, SCH, K)
    col4 = ei[1].reshape(NW, NSEC, SCH, K)
    agg, cnt = _sc_aggregate(x, row4, col4)
    cnt3 = cnt.reshape(NC, NPAD, 1)
    out = pl.pallas_call(
        _tc_body,
        grid=(2, NB),
        in_specs=[
            pl.BlockSpec((BR, D), lambda p, i: (i, 0)),
            pl.BlockSpec((NC, BR, D), lambda p, i: (0, i, 0)),
            pl.BlockSpec((NC, BR, 1), lambda p, i: (0, i, 0)),
            pl.BlockSpec((D, D), lambda p, i: (0, 0)),
            pl.BlockSpec((1, D), lambda p, i: (0, 0)),
            pl.BlockSpec((1, D), lambda p, i: (0, 0)),
            pl.BlockSpec((1, D), lambda p, i: (0, 0)),
        ],
        out_specs=pl.BlockSpec((BR, D), lambda p, i: (i, 0)),
        scratch_shapes=[
            pltpu.VMEM((N, D), jnp.float32),
            pltpu.VMEM((1, D), jnp.float32),
            pltpu.VMEM((1, D), jnp.float32),
        ],
        out_shape=jax.ShapeDtypeStruct((N, D), jnp.float32),
    )(x, agg, cnt3, W, b.reshape(1, D), gamma.reshape(1, D),
      beta.reshape(1, D))
    return out


# speedup vs baseline: 1.2287x; 1.0831x over previous
import functools

import jax
import jax.numpy as jnp
from jax import lax
from jax.experimental import pallas as pl
from jax.experimental.pallas import tpu as pltpu
from jax.experimental.pallas import tpu_sc as plsc

N = 10000
E = 320000
D = 128
NC, NS = 2, 16
NW = NC * NS
EPW = E // NW
K = 100
NCHUNK = EPW // K
SCH = 50
NSEC = NCHUNK // SCH
NPAD = 10240
RPT = NPAD // NS
ZB = 80


def _sc_aggregate(x, row4, col4):
    mesh = plsc.VectorSubcoreMesh(core_axis_name="c", subcore_axis_name="s")

    @functools.partial(
        pl.kernel,
        out_type=(
            jax.ShapeDtypeStruct((NC, NPAD, D), jnp.float32),
            jax.ShapeDtypeStruct((NC, NPAD), jnp.float32),
        ),
        mesh=mesh,
        scratch_types=[
            pltpu.VMEM_SHARED((NPAD, D), jnp.float32),
            pltpu.VMEM_SHARED((NPAD,), jnp.float32),
            pltpu.VMEM((SCH, K), jnp.int32),
            pltpu.VMEM((SCH, K), jnp.int32),
            pltpu.VMEM((K, D), jnp.float32),
            pltpu.VMEM((K, D), jnp.float32),
            pltpu.VMEM((K,), jnp.float32),
            pltpu.VMEM((RPT,), jnp.float32),
            pltpu.SemaphoreType.DMA,
            pltpu.SemaphoreType.DMA,
        ],
    )
    def agg_kernel(x_hbm, row_hbm, col_hbm, ones_hbm, agg_hbm, cnt_hbm,
                   agg_s, cnt_s, ridx, cidx, rows0, rows1, ones, zcnt,
                   sem0, sem1):
        c = lax.axis_index("c")
        s = lax.axis_index("s")
        wid = s * NC + c

        zv = jnp.zeros((16,), jnp.float32)

        pltpu.sync_copy(ones_hbm, ones)

        def _zc(i, carry):
            zcnt[pl.ds(i * 16, 16)] = zv
            return carry
        lax.fori_loop(0, RPT // 16, _zc, 0)

        def _zb(i, carry):
            for j in range(D // 16):
                rows0[i, pl.ds(j * 16, 16)] = zv
            return carry
        lax.fori_loop(0, K, _zb, 0)

        base_r = s * RPT
        for jb in range(RPT // ZB):
            pltpu.sync_copy(rows0.at[pl.ds(0, ZB), :],
                            agg_s.at[pl.ds(base_r + jb * ZB, ZB), :])
        pltpu.sync_copy(zcnt, cnt_s.at[pl.ds(base_r, RPT)])

        plsc.subcore_barrier()

        def _g_start(j, buf, sem):
            pltpu.async_copy(x_hbm.at[cidx.at[j]], buf, sem)

        def _g_wait(j, buf, sem):
            pltpu.make_async_copy(x_hbm.at[cidx.at[j]], buf, sem).wait()

        def _scat(j, buf):
            pltpu.sync_copy(buf, agg_s.at[ridx.at[j]], add=True)
            pltpu.sync_copy(ones, cnt_s.at[ridx.at[j]], add=True)

        def _sec(sec, carry):
            pltpu.sync_copy(row_hbm.at[wid, sec], ridx)
            pltpu.sync_copy(col_hbm.at[wid, sec], cidx)

            _g_start(0, rows0, sem0)

            def _pair(t, carry2):
                j0 = t * 2
                _g_start(j0 + 1, rows1, sem1)
                _g_wait(j0, rows0, sem0)
                _scat(j0, rows0)
                _g_start(j0 + 2, rows0, sem0)
                _g_wait(j0 + 1, rows1, sem1)
                _scat(j0 + 1, rows1)
                return carry2
            lax.fori_loop(0, (SCH - 2) // 2, _pair, 0)

            _g_start(SCH - 1, rows1, sem1)
            _g_wait(SCH - 2, rows0, sem0)
            _scat(SCH - 2, rows0)
            _g_wait(SCH - 1, rows1, sem1)
            _scat(SCH - 1, rows1)
            return carry
        lax.fori_loop(0, NSEC, _sec, 0)

        plsc.subcore_barrier()

        pltpu.sync_copy(agg_s.at[pl.ds(base_r, RPT), :],
                        agg_hbm.at[c, pl.ds(base_r, RPT), :])
        pltpu.sync_copy(cnt_s.at[pl.ds(base_r, RPT)],
                        cnt_hbm.at[c, pl.ds(base_r, RPT)])

    return agg_kernel(x, row4, col4, jnp.ones((K,), jnp.float32))


def _tc_body(x_ref, agg_ref, cnt_ref, w_ref, b_ref, g_ref, be_ref, out_ref):
    x = x_ref[...]
    agg = agg_ref[0, :N, :] + agg_ref[1, :N, :]
    cnt = cnt_ref[0, :N, :] + cnt_ref[1, :N, :]
    inv = 1.0 / (cnt + 1e-8)
    u = x + agg * inv
    y = lax.dot_general(u, w_ref[...], (((1,), (1,)), ((), ())),
                        preferred_element_type=jnp.float32)
    y = y + (1.0 + cnt * inv) * b_ref[...]
    mean = jnp.mean(y, axis=0, keepdims=True)
    yc = y - mean
    var = jnp.mean(yc * yc, axis=0, keepdims=True)
    out_ref[...] = yc * lax.rsqrt(var + 1e-5) * g_ref[...] + be_ref[...]


def kernel(x, edge_index, batch_size, W, b, gamma, beta):
    del batch_size
    ei = edge_index.astype(jnp.int32)
    row4 = ei[0].reshape(NW, NSEC, SCH, K)
    col4 = ei[1].reshape(NW, NSEC, SCH, K)
    agg, cnt = _sc_aggregate(x, row4, col4)
    cnt3 = cnt.reshape(NC, NPAD, 1)
    out = pl.pallas_call(
        _tc_body,
        out_shape=jax.ShapeDtypeStruct((N, D), jnp.float32),
    )(x, agg, cnt3, W, b.reshape(1, D), gamma.reshape(1, D),
      beta.reshape(1, D))
    return out
